# speedup vs baseline: 3.0933x; 3.0933x over previous
import functools

import jax
import jax.numpy as jnp
from jax import lax
from jax.experimental import pallas as pl
from jax.experimental.pallas import tpu as pltpu
from jax.experimental.pallas import tpu_sc as plsc

N = 10000
D = 128
R = 6
E = 50000
RE = R * E
CH = 120
NCHUNKS = RE // CH
MLP_BLK = 2400
MLP_STEPS = RE // MLP_BLK


def _leaky(x):
    return jnp.where(x >= 0, x, 0.01 * x)


def _pre_body(ne_ref, ge_ref, de_ref, dr_ref, a_ref, b_ref):
    a_ref[0] = jnp.dot(ne_ref[...], ge_ref[0], preferred_element_type=jnp.float32)
    b_ref[0] = jnp.dot(de_ref[...], dr_ref[0], preferred_element_type=jnp.float32)


def _precompute(nodes_emb, gen_rel, dis_node_emb, dis_rel):
    return pl.pallas_call(
        _pre_body,
        grid=(R,),
        in_specs=[
            pl.BlockSpec((N, D), lambda r: (0, 0)),
            pl.BlockSpec((1, D, D), lambda r: (r, 0, 0)),
            pl.BlockSpec((N, D), lambda r: (0, 0)),
            pl.BlockSpec((1, D, D), lambda r: (r, 0, 0)),
        ],
        out_specs=[
            pl.BlockSpec((1, N, D), lambda r: (r, 0, 0)),
            pl.BlockSpec((1, N, D), lambda r: (r, 0, 0)),
        ],
        out_shape=[
            jax.ShapeDtypeStruct((R, N, D), jnp.float32),
            jax.ShapeDtypeStruct((R, N, D), jnp.float32),
        ],
    )(nodes_emb, gen_rel, dis_node_emb, dis_rel)


def _make_gather():
    info = plsc.get_sparse_core_info()
    nc, ns = info.num_cores, info.num_subcores
    nw = nc * ns
    base_chunks, rem_chunks = NCHUNKS // nw, NCHUNKS % nw
    mesh = plsc.VectorSubcoreMesh(core_axis_name="c", subcore_axis_name="s")

    @functools.partial(
        pl.kernel,
        mesh=mesh,
        out_type=[
            jax.ShapeDtypeStruct((RE, D), jnp.float32),
            jax.ShapeDtypeStruct((RE, D), jnp.float32),
        ],
        scratch_types=[
            pltpu.VMEM((CH,), jnp.int32),
            pltpu.VMEM((CH, D), jnp.float32),
            pltpu.VMEM((CH, D), jnp.float32),
            pltpu.SemaphoreType.DMA,
            pltpu.SemaphoreType.DMA,
        ],
    )
    def gather_k(ta, tb, idx, out_a, out_b, idx_v, rows_a, rows_b, sem_a, sem_b):
        wid = lax.axis_index("s") * nc + lax.axis_index("c")
        n_mine = base_chunks + jnp.where(wid < rem_chunks, 1, 0)

        def body(j, carry):
            c = wid + j * nw
            base = c * CH
            pltpu.sync_copy(idx.at[pl.ds(base, CH)], idx_v)
            cp_a = pltpu.async_copy(ta.at[idx_v], rows_a, sem_a)
            cp_b = pltpu.async_copy(tb.at[idx_v], rows_b, sem_b)
            cp_a.wait()
            cp_b.wait()
            pltpu.sync_copy(rows_a, out_a.at[pl.ds(base, CH)])
            pltpu.sync_copy(rows_b, out_b.at[pl.ds(base, CH)])
            return carry

        lax.fori_loop(0, n_mine, body, 0)

    return gather_k


def _mlp_body(ga_ref, gb_ref, nz_ref, w1_ref, b1_ref, w2_ref, b2_ref, out_ref):
    x = ga_ref[...] + nz_ref[...]
    h = lax.dot_general(x, w1_ref[...], (((1,), (1,)), ((), ())),
                        preferred_element_type=jnp.float32) + b1_ref[...]
    h = _leaky(h)
    h = lax.dot_general(h, w2_ref[...], (((1,), (1,)), ((), ())),
                        preferred_element_type=jnp.float32) + b2_ref[...]
    h = _leaky(h)
    out_ref[0, 0, :] = jnp.sum(gb_ref[...] * h, axis=1)


def _mlp_score(ga, gb, noise, w1, b1, w2, b2):
    out = pl.pallas_call(
        _mlp_body,
        grid=(MLP_STEPS,),
        in_specs=[
            pl.BlockSpec((MLP_BLK, D), lambda i: (i, 0)),
            pl.BlockSpec((MLP_BLK, D), lambda i: (i, 0)),
            pl.BlockSpec((MLP_BLK, D), lambda i: (i, 0)),
            pl.BlockSpec((D, D), lambda i: (0, 0)),
            pl.BlockSpec((D,), lambda i: (0,)),
            pl.BlockSpec((D, D), lambda i: (0, 0)),
            pl.BlockSpec((D,), lambda i: (0,)),
        ],
        out_specs=pl.BlockSpec((1, 1, MLP_BLK), lambda i: (i, 0, 0)),
        out_shape=jax.ShapeDtypeStruct((MLP_STEPS, 1, MLP_BLK), jnp.float32),
    )(ga, gb, noise, w1, b1, w2, b2)
    return out.reshape(-1)


def kernel(dis_node_emb, dis_relation_matrix, noise_emb, edge_src,
           nodes_emb, gen_relation_matrix, W1, b1, W2, b2):
    a_tab, b_tab = _precompute(nodes_emb, gen_relation_matrix,
                               dis_node_emb, dis_relation_matrix)
    ta = a_tab.reshape(R * N, D)
    tb = b_tab.reshape(R * N, D)
    adj_idx = (edge_src
               + (jnp.arange(R, dtype=jnp.int32) * N)[:, None]).reshape(-1)
    ga, gb = _make_gather()(ta, tb, adj_idx)
    noise = noise_emb.reshape(RE, D)
    return _mlp_score(ga, gb, noise, W1, b1, W2, b2)


# speedup vs baseline: 3.4503x; 1.1154x over previous
import functools

import jax
import jax.numpy as jnp
from jax import lax
from jax.experimental import pallas as pl
from jax.experimental.pallas import tpu as pltpu
from jax.experimental.pallas import tpu_sc as plsc

N = 10000
D = 128
R = 6
E = 50000
RE = R * E
CH = 120
NCHUNKS = RE // CH
MLP_BLK = 2400
MLP_STEPS = RE // MLP_BLK


def _leaky(x):
    return jnp.where(x >= 0, x, 0.01 * x)


def _pre_body(ne_ref, ge_ref, de_ref, dr_ref, a_ref, b_ref):
    a_ref[0] = jnp.dot(ne_ref[...], ge_ref[0], preferred_element_type=jnp.float32)
    b_ref[0] = jnp.dot(de_ref[...], dr_ref[0], preferred_element_type=jnp.float32)


def _precompute(nodes_emb, gen_rel, dis_node_emb, dis_rel):
    return pl.pallas_call(
        _pre_body,
        grid=(R,),
        in_specs=[
            pl.BlockSpec((N, D), lambda r: (0, 0)),
            pl.BlockSpec((1, D, D), lambda r: (r, 0, 0)),
            pl.BlockSpec((N, D), lambda r: (0, 0)),
            pl.BlockSpec((1, D, D), lambda r: (r, 0, 0)),
        ],
        out_specs=[
            pl.BlockSpec((1, N, D), lambda r: (r, 0, 0)),
            pl.BlockSpec((1, N, D), lambda r: (r, 0, 0)),
        ],
        out_shape=[
            jax.ShapeDtypeStruct((R, N, D), jnp.float32),
            jax.ShapeDtypeStruct((R, N, D), jnp.float32),
        ],
    )(nodes_emb, gen_rel, dis_node_emb, dis_rel)


def _make_gather():
    info = plsc.get_sparse_core_info()
    nc, ns = info.num_cores, info.num_subcores
    nw = nc * ns
    trip = -(-NCHUNKS // nw)
    trip_pad = trip + (trip % 2)
    mesh = plsc.VectorSubcoreMesh(core_axis_name="c", subcore_axis_name="s")

    @functools.partial(
        pl.kernel,
        mesh=mesh,
        out_type=[
            jax.ShapeDtypeStruct((RE, D), jnp.float32),
            jax.ShapeDtypeStruct((RE, D), jnp.float32),
        ],
        scratch_types=[
            pltpu.VMEM((2, CH), jnp.int32),
            pltpu.VMEM((2, CH, D), jnp.float32),
            pltpu.VMEM((2, CH, D), jnp.float32),
            pltpu.SemaphoreType.DMA,
            pltpu.SemaphoreType.DMA,
            pltpu.SemaphoreType.DMA,
            pltpu.SemaphoreType.DMA,
        ],
    )
    def gather_k(ta, tb, idx, out_a, out_b, idx_v, rows_a, rows_b,
                 sem_a0, sem_a1, sem_b0, sem_b1):
        wid = lax.axis_index("s") * nc + lax.axis_index("c")
        sems_a = (sem_a0, sem_a1)
        sems_b = (sem_b0, sem_b1)

        def start(j, b):
            c = wid + j * nw

            @pl.when(c < NCHUNKS)
            def _():
                base = c * CH
                pltpu.sync_copy(idx.at[pl.ds(base, CH)], idx_v.at[b])
                pltpu.async_copy(ta.at[idx_v.at[b]], rows_a.at[b], sems_a[b])
                pltpu.async_copy(tb.at[idx_v.at[b]], rows_b.at[b], sems_b[b])

        def finish(j, b):
            c = wid + j * nw

            @pl.when(c < NCHUNKS)
            def _():
                base = c * CH
                pltpu.make_async_copy(ta.at[idx_v.at[b]], rows_a.at[b],
                                      sems_a[b]).wait()
                pltpu.make_async_copy(tb.at[idx_v.at[b]], rows_b.at[b],
                                      sems_b[b]).wait()
                pltpu.sync_copy(rows_a.at[b], out_a.at[pl.ds(base, CH)])
                pltpu.sync_copy(rows_b.at[b], out_b.at[pl.ds(base, CH)])

        start(0, 0)

        def body(i, carry):
            o = 2 * i
            start(o + 1, 1)
            finish(o, 0)
            start(o + 2, 0)
            finish(o + 1, 1)
            return carry

        lax.fori_loop(0, trip_pad // 2, body, 0)

    return gather_k


def _mlp_body(ga_ref, gb_ref, nz_ref, w1_ref, b1_ref, w2_ref, b2_ref, out_ref):
    x = ga_ref[...] + nz_ref[...]
    h = lax.dot_general(x, w1_ref[...], (((1,), (1,)), ((), ())),
                        preferred_element_type=jnp.float32) + b1_ref[...]
    h = _leaky(h)
    h = lax.dot_general(h, w2_ref[...], (((1,), (1,)), ((), ())),
                        preferred_element_type=jnp.float32) + b2_ref[...]
    h = _leaky(h)
    out_ref[0, 0, :] = jnp.sum(gb_ref[...] * h, axis=1)


def _mlp_score(ga, gb, noise, w1, b1, w2, b2):
    out = pl.pallas_call(
        _mlp_body,
        grid=(MLP_STEPS,),
        in_specs=[
            pl.BlockSpec((MLP_BLK, D), lambda i: (i, 0)),
            pl.BlockSpec((MLP_BLK, D), lambda i: (i, 0)),
            pl.BlockSpec((MLP_BLK, D), lambda i: (i, 0)),
            pl.BlockSpec((D, D), lambda i: (0, 0)),
            pl.BlockSpec((D,), lambda i: (0,)),
            pl.BlockSpec((D, D), lambda i: (0, 0)),
            pl.BlockSpec((D,), lambda i: (0,)),
        ],
        out_specs=pl.BlockSpec((1, 1, MLP_BLK), lambda i: (i, 0, 0)),
        out_shape=jax.ShapeDtypeStruct((MLP_STEPS, 1, MLP_BLK), jnp.float32),
    )(ga, gb, noise, w1, b1, w2, b2)
    return out.reshape(-1)


def kernel(dis_node_emb, dis_relation_matrix, noise_emb, edge_src,
           nodes_emb, gen_relation_matrix, W1, b1, W2, b2):
    a_tab, b_tab = _precompute(nodes_emb, gen_relation_matrix,
                               dis_node_emb, dis_relation_matrix)
    ta = a_tab.reshape(R * N, D)
    tb = b_tab.reshape(R * N, D)
    adj_idx = (edge_src
               + (jnp.arange(R, dtype=jnp.int32) * N)[:, None]).reshape(-1)
    ga, gb = _make_gather()(ta, tb, adj_idx)
    noise = noise_emb.reshape(RE, D)
    return _mlp_score(ga, gb, noise, W1, b1, W2, b2)


# speedup vs baseline: 5.7975x; 1.6803x over previous
import functools

import jax
import jax.numpy as jnp
from jax import lax
from jax.experimental import pallas as pl
from jax.experimental.pallas import tpu as pltpu
from jax.experimental.pallas import tpu_sc as plsc

N = 10000
D = 128
H = D // 2
R = 6
E = 50000
RE = R * E
CH = 120
NCHUNKS = RE // CH
MLP_BLK = 2400
MLP_STEPS = RE // MLP_BLK

_HI = -65536
_LO = 0xFFFF


def _leaky(x):
    return jnp.where(x >= 0, x, 0.01 * x)


def _rnd_bf16_bits(f):
    bits = lax.bitcast_convert_type(f, jnp.int32)
    return bits + 0x7FFF + ((bits >> 16) & 1)


def _pack_halves(a):
    lo = (_rnd_bf16_bits(a[:, 0:H]) >> 16) & _LO
    hi = _rnd_bf16_bits(a[:, H:D]) & _HI
    return hi | lo


def _pre_body(ne_ref, ge_ref, de_ref, dr_ref, c_ref):
    a = jnp.dot(ne_ref[...], ge_ref[0], preferred_element_type=jnp.float32)
    b = jnp.dot(de_ref[...], dr_ref[0], preferred_element_type=jnp.float32)
    c_ref[0, :, 0:H] = _pack_halves(a)
    c_ref[0, :, H:D] = _pack_halves(b)


def _precompute(nodes_emb, gen_rel, dis_node_emb, dis_rel):
    return pl.pallas_call(
        _pre_body,
        grid=(R,),
        in_specs=[
            pl.BlockSpec((N, D), lambda r: (0, 0)),
            pl.BlockSpec((1, D, D), lambda r: (r, 0, 0)),
            pl.BlockSpec((N, D), lambda r: (0, 0)),
            pl.BlockSpec((1, D, D), lambda r: (r, 0, 0)),
        ],
        out_specs=pl.BlockSpec((1, N, D), lambda r: (r, 0, 0)),
        out_shape=jax.ShapeDtypeStruct((R, N, D), jnp.int32),
    )(nodes_emb, gen_rel, dis_node_emb, dis_rel)


def _make_gather():
    info = plsc.get_sparse_core_info()
    nc, ns = info.num_cores, info.num_subcores
    nw = nc * ns
    trip = -(-NCHUNKS // nw)
    trip_pad = trip + (trip % 2)
    mesh = plsc.VectorSubcoreMesh(core_axis_name="c", subcore_axis_name="s")

    @functools.partial(
        pl.kernel,
        mesh=mesh,
        out_type=jax.ShapeDtypeStruct((RE, D), jnp.int32),
        scratch_types=[
            pltpu.VMEM((2, CH), jnp.int32),
            pltpu.VMEM((2, CH, D), jnp.int32),
            pltpu.SemaphoreType.DMA,
            pltpu.SemaphoreType.DMA,
        ],
    )
    def gather_k(tab, idx, out, idx_v, rows_v, sem0, sem1):
        wid = lax.axis_index("s") * nc + lax.axis_index("c")
        sems = (sem0, sem1)

        def start(j, b):
            c = wid + j * nw

            @pl.when(c < NCHUNKS)
            def _():
                base = c * CH
                pltpu.sync_copy(idx.at[pl.ds(base, CH)], idx_v.at[b])
                pltpu.async_copy(tab.at[idx_v.at[b]], rows_v.at[b], sems[b])

        def finish(j, b):
            c = wid + j * nw

            @pl.when(c < NCHUNKS)
            def _():
                base = c * CH
                pltpu.make_async_copy(tab.at[idx_v.at[b]], rows_v.at[b],
                                      sems[b]).wait()
                pltpu.sync_copy(rows_v.at[b], out.at[pl.ds(base, CH)])

        start(0, 0)

        def body(i, carry):
            o = 2 * i
            start(o + 1, 1)
            finish(o, 0)
            start(o + 2, 0)
            finish(o + 1, 1)
            return carry

        lax.fori_loop(0, trip_pad // 2, body, 0)

    return gather_k


def _mlp_body(gab_ref, nz_ref, w1_ref, b1_ref, w2_ref, b2_ref, out_ref):
    xi = gab_ref[...]
    lo_f = lax.bitcast_convert_type(xi << 16, jnp.float32)
    hi_f = lax.bitcast_convert_type(xi & _HI, jnp.float32)
    a = jnp.concatenate([lo_f[:, 0:H], hi_f[:, 0:H]], axis=1)
    b = jnp.concatenate([lo_f[:, H:D], hi_f[:, H:D]], axis=1)
    x = a + nz_ref[...]
    h = lax.dot_general(x, w1_ref[...], (((1,), (1,)), ((), ())),
                        preferred_element_type=jnp.float32) + b1_ref[...]
    h = _leaky(h)
    h = lax.dot_general(h, w2_ref[...], (((1,), (1,)), ((), ())),
                        preferred_element_type=jnp.float32) + b2_ref[...]
    h = _leaky(h)
    out_ref[0, 0, :] = jnp.sum(b * h, axis=1)


def _mlp_score(gab, noise, w1, b1, w2, b2):
    out = pl.pallas_call(
        _mlp_body,
        grid=(MLP_STEPS,),
        in_specs=[
            pl.BlockSpec((MLP_BLK, D), lambda i: (i, 0)),
            pl.BlockSpec((MLP_BLK, D), lambda i: (i, 0)),
            pl.BlockSpec((D, D), lambda i: (0, 0)),
            pl.BlockSpec((D,), lambda i: (0,)),
            pl.BlockSpec((D, D), lambda i: (0, 0)),
            pl.BlockSpec((D,), lambda i: (0,)),
        ],
        out_specs=pl.BlockSpec((1, 1, MLP_BLK), lambda i: (i, 0, 0)),
        out_shape=jax.ShapeDtypeStruct((MLP_STEPS, 1, MLP_BLK), jnp.float32),
    )(gab, noise, w1, b1, w2, b2)
    return out.reshape(-1)


def kernel(dis_node_emb, dis_relation_matrix, noise_emb, edge_src,
           nodes_emb, gen_relation_matrix, W1, b1, W2, b2):
    c_tab = _precompute(nodes_emb, gen_relation_matrix,
                        dis_node_emb, dis_relation_matrix)
    tab = c_tab.reshape(R * N, D)
    adj_idx = (edge_src
               + (jnp.arange(R, dtype=jnp.int32) * N)[:, None]).reshape(-1)
    gab = _make_gather()(tab, adj_idx)
    noise = noise_emb.reshape(RE, D)
    return _mlp_score(gab, noise, W1, b1, W2, b2)


# speedup vs baseline: 6.4691x; 1.1158x over previous
import functools

import jax
import jax.numpy as jnp
from jax import lax
from jax.experimental import pallas as pl
from jax.experimental.pallas import tpu as pltpu
from jax.experimental.pallas import tpu_sc as plsc

N = 10000
D = 128
H = D // 2
R = 6
E = 50000
RE = R * E
CH = 120
NSLICE = 5
SLICE = RE // NSLICE
NCHUNKS = SLICE // CH
MLP_BLK = 2400
MLP_STEPS = SLICE // MLP_BLK

_HI = -65536
_LO = 0xFFFF


def _leaky(x):
    return jnp.where(x >= 0, x, 0.01 * x)


def _rnd_bf16_bits(f):
    bits = lax.bitcast_convert_type(f, jnp.int32)
    return bits + 0x7FFF + ((bits >> 16) & 1)


def _pack_halves(a):
    lo = (_rnd_bf16_bits(a[:, 0:H]) >> 16) & _LO
    hi = _rnd_bf16_bits(a[:, H:D]) & _HI
    return hi | lo


def _pre_body(ne_ref, ge_ref, de_ref, dr_ref, c_ref):
    a = jnp.dot(ne_ref[...], ge_ref[0], preferred_element_type=jnp.float32)
    b = jnp.dot(de_ref[...], dr_ref[0], preferred_element_type=jnp.float32)
    c_ref[0, :, 0:H] = _pack_halves(a)
    c_ref[0, :, H:D] = _pack_halves(b)


def _precompute(nodes_emb, gen_rel, dis_node_emb, dis_rel):
    return pl.pallas_call(
        _pre_body,
        grid=(R,),
        in_specs=[
            pl.BlockSpec((N, D), lambda r: (0, 0)),
            pl.BlockSpec((1, D, D), lambda r: (r, 0, 0)),
            pl.BlockSpec((N, D), lambda r: (0, 0)),
            pl.BlockSpec((1, D, D), lambda r: (r, 0, 0)),
        ],
        out_specs=pl.BlockSpec((1, N, D), lambda r: (r, 0, 0)),
        out_shape=jax.ShapeDtypeStruct((R, N, D), jnp.int32),
    )(nodes_emb, gen_rel, dis_node_emb, dis_rel)


def _make_gather():
    info = plsc.get_sparse_core_info()
    nc, ns = info.num_cores, info.num_subcores
    nw = nc * ns
    trip = -(-NCHUNKS // nw)
    trip_pad = trip + (trip % 2)
    mesh = plsc.VectorSubcoreMesh(core_axis_name="c", subcore_axis_name="s")

    @functools.partial(
        pl.kernel,
        mesh=mesh,
        out_type=jax.ShapeDtypeStruct((SLICE, D), jnp.int32),
        scratch_types=[
            pltpu.VMEM((2, CH), jnp.int32),
            pltpu.VMEM((2, CH, D), jnp.int32),
            pltpu.SemaphoreType.DMA,
            pltpu.SemaphoreType.DMA,
        ],
    )
    def gather_k(tab, idx, out, idx_v, rows_v, sem0, sem1):
        wid = lax.axis_index("s") * nc + lax.axis_index("c")
        sems = (sem0, sem1)

        def start(j, b):
            c = wid + j * nw

            @pl.when(c < NCHUNKS)
            def _():
                base = c * CH
                pltpu.sync_copy(idx.at[pl.ds(base, CH)], idx_v.at[b])
                pltpu.async_copy(tab.at[idx_v.at[b]], rows_v.at[b], sems[b])

        def finish(j, b):
            c = wid + j * nw

            @pl.when(c < NCHUNKS)
            def _():
                base = c * CH
                pltpu.make_async_copy(tab.at[idx_v.at[b]], rows_v.at[b],
                                      sems[b]).wait()
                pltpu.sync_copy(rows_v.at[b], out.at[pl.ds(base, CH)])

        start(0, 0)

        def body(i, carry):
            o = 2 * i
            start(o + 1, 1)
            finish(o, 0)
            start(o + 2, 0)
            finish(o + 1, 1)
            return carry

        lax.fori_loop(0, trip_pad // 2, body, 0)

    return gather_k


def _mlp_body(gab_ref, nz_ref, w1_ref, b1_ref, w2_ref, b2_ref, out_ref):
    xi = gab_ref[...]
    lo_f = lax.bitcast_convert_type(xi << 16, jnp.float32)
    hi_f = lax.bitcast_convert_type(xi & _HI, jnp.float32)
    a = jnp.concatenate([lo_f[:, 0:H], hi_f[:, 0:H]], axis=1)
    b = jnp.concatenate([lo_f[:, H:D], hi_f[:, H:D]], axis=1)
    x = a + nz_ref[...]
    h = lax.dot_general(x.astype(jnp.bfloat16), w1_ref[...],
                        (((1,), (1,)), ((), ())),
                        preferred_element_type=jnp.float32) + b1_ref[...]
    h = _leaky(h)
    h = lax.dot_general(h.astype(jnp.bfloat16), w2_ref[...],
                        (((1,), (1,)), ((), ())),
                        preferred_element_type=jnp.float32) + b2_ref[...]
    h = _leaky(h)
    out_ref[0, 0, :] = jnp.sum(b * h, axis=1)


def _mlp_score(gab_s, noise, w1, b1, w2, b2, s):
    off = s * MLP_STEPS
    out = pl.pallas_call(
        _mlp_body,
        grid=(MLP_STEPS,),
        in_specs=[
            pl.BlockSpec((MLP_BLK, D), lambda i: (i, 0)),
            pl.BlockSpec((MLP_BLK, D), lambda i: (i + off, 0)),
            pl.BlockSpec((D, D), lambda i: (0, 0)),
            pl.BlockSpec((D,), lambda i: (0,)),
            pl.BlockSpec((D, D), lambda i: (0, 0)),
            pl.BlockSpec((D,), lambda i: (0,)),
        ],
        out_specs=pl.BlockSpec((1, 1, MLP_BLK), lambda i: (i, 0, 0)),
        out_shape=jax.ShapeDtypeStruct((MLP_STEPS, 1, MLP_BLK), jnp.float32),
    )(gab_s, noise, w1, b1, w2, b2)
    return out.reshape(-1)


def kernel(dis_node_emb, dis_relation_matrix, noise_emb, edge_src,
           nodes_emb, gen_relation_matrix, W1, b1, W2, b2):
    c_tab = _precompute(nodes_emb, gen_relation_matrix,
                        dis_node_emb, dis_relation_matrix)
    tab = c_tab.reshape(R * N, D)
    adj_idx = (edge_src
               + (jnp.arange(R, dtype=jnp.int32) * N)[:, None]).reshape(-1)
    noise = noise_emb.reshape(RE, D)
    w1b = W1.astype(jnp.bfloat16)
    w2b = W2.astype(jnp.bfloat16)
    gather = _make_gather()
    scores = []
    for s in range(NSLICE):
        gab_s = gather(tab, lax.slice(adj_idx, (s * SLICE,), ((s + 1) * SLICE,)))
        scores.append(_mlp_score(gab_s, noise, w1b, b1, w2b, b2, s))
    return jnp.concatenate(scores)
